# Initial kernel scaffold; baseline (speedup 1.0000x reference)
import functools

import jax
import jax.numpy as jnp
import numpy as np
from jax import lax
from jax.experimental import pallas as pl
from jax.experimental.pallas import tpu as pltpu
from jax.experimental.pallas import tpu_sc as plsc

_NUM_BUCKETS = 128
_NUM_HEADS = 8
_MAX_TIME_DIFF = 2592000.0
_B = 4096
_S = 50
_P = _S * _S

_SCALE = (_NUM_BUCKETS - 1) / np.log(_MAX_TIME_DIFF + 1.0)
_L = 2.0 / np.log(2.0)
_C0 = np.float32(_L)
_C1 = np.float32(_L / 3.0)
_C2 = np.float32(_L / 5.0)
_C3 = np.float32(_L / 7.0)
_C4 = np.float32(_L / 9.0)
_SQRT2 = np.float32(np.sqrt(2.0))
_LN2_SCALE = np.float32(np.log(2.0) * _SCALE)

_INFO = plsc.get_sparse_core_info()
_NW = _INFO.num_cores * _INFO.num_subcores
_ITEMS_PER_W = _B // _NW

_N_CHUNKS = 157
_TAIL_START = _P - 16


def _bucket_ids(d):
    y = d + jnp.float32(1.0)
    yi = lax.bitcast_convert_type(y, jnp.int32)
    e = lax.shift_right_arithmetic(yi, jnp.int32(23)) - jnp.int32(127)
    m = lax.bitcast_convert_type(
        (yi & jnp.int32(0x7FFFFF)) | jnp.int32(0x3F800000), jnp.float32)
    big = m > _SQRT2
    m = jnp.where(big, m * jnp.float32(0.5), m)
    ef = (e + big.astype(jnp.int32)).astype(jnp.float32)
    s = (m - jnp.float32(1.0)) / (m + jnp.float32(1.0))
    z = s * s
    p = _C4
    p = p * z + _C3
    p = p * z + _C2
    p = p * z + _C1
    p = p * z + _C0
    v = (ef + s * p) * _LN2_SCALE
    idx = v.astype(jnp.int32)
    return jnp.clip(idx, 0, _NUM_BUCKETS - 1)


def _sc_kernel(raw_hbm, tab_hbm, out_hbm, rawv, tabv, tbuf, outbuf):
    wid = lax.axis_index("s") * _INFO.num_cores + lax.axis_index("c")
    b0 = wid * _ITEMS_PER_W

    pltpu.sync_copy(tab_hbm, tabv)
    pltpu.sync_copy(raw_hbm.at[pl.ds(b0 * _S, _ITEMS_PER_W * _S)], rawv)

    iota = lax.iota(jnp.int32, 16)

    def item_body(k, carry):
        base = k * _S
        ch0 = plsc.load_gather(rawv, [base + iota])
        ch1 = plsc.load_gather(rawv, [base + 16 + iota])
        ch2 = plsc.load_gather(rawv, [base + 32 + iota])
        ch3 = plsc.load_gather(rawv, [jnp.minimum(base + 48 + iota,
                                                  jnp.int32(_ITEMS_PER_W * _S - 1))])
        t0 = plsc.cumsum(ch0)
        s0 = jnp.sum(ch0)
        t1 = plsc.cumsum(ch1) + s0
        s1 = s0 + jnp.sum(ch1)
        t2 = plsc.cumsum(ch2) + s1
        s2 = s1 + jnp.sum(ch2)
        t3 = plsc.cumsum(ch3) + s2
        tbuf[pl.ds(0, 16)] = t0
        tbuf[pl.ds(16, 16)] = t1
        tbuf[pl.ds(32, 16)] = t2
        tbuf[pl.ds(48, 16)] = t3

        def chunk_body(c, carry2):
            pbase = jnp.minimum(c * 16, jnp.int32(_TAIL_START))
            pv = pbase + iota
            iv = pv // jnp.int32(_S)
            jv = pv - iv * jnp.int32(_S)
            ti = plsc.load_gather(tbuf, [iv])
            tj = plsc.load_gather(tbuf, [jv])
            d = jnp.abs(ti - tj)
            idx8 = lax.shift_left(_bucket_ids(d), jnp.int32(3))
            for h in range(_NUM_HEADS):
                g = plsc.load_gather(tabv, [idx8 + jnp.int32(h)])
                plsc.store_scatter(outbuf, [pv + jnp.int32(h * _P)], g)
            return carry2

        lax.fori_loop(0, _N_CHUNKS, chunk_body, 0, unroll=2)
        pltpu.sync_copy(outbuf, out_hbm.at[b0 + k])
        return carry

    lax.fori_loop(0, _ITEMS_PER_W, item_body, 0)


def kernel(raw_time_diff, time_emb_weight):
    tab_flat = time_emb_weight.reshape(_NUM_BUCKETS * _NUM_HEADS)
    raw_flat = raw_time_diff.reshape(_B * _S)

    mesh = plsc.VectorSubcoreMesh(core_axis_name="c", subcore_axis_name="s")
    run = functools.partial(
        pl.kernel,
        mesh=mesh,
        out_type=jax.ShapeDtypeStruct((_B, _NUM_HEADS * _P), jnp.float32),
        scratch_types=[
            pltpu.VMEM((_ITEMS_PER_W * _S,), jnp.float32),
            pltpu.VMEM((_NUM_BUCKETS * _NUM_HEADS,), jnp.float32),
            pltpu.VMEM((64,), jnp.float32),
            pltpu.VMEM((_NUM_HEADS * _P,), jnp.float32),
        ],
    )(_sc_kernel)
    out = run(raw_flat, tab_flat)
    return out.reshape(_B, _NUM_HEADS, _S, _S)


# speedup vs baseline: 18.0304x; 18.0304x over previous
import functools

import jax
import jax.numpy as jnp
import numpy as np
from jax import lax
from jax.experimental import pallas as pl
from jax.experimental.pallas import tpu as pltpu
from jax.experimental.pallas import tpu_sc as plsc

_NUM_BUCKETS = 128
_NUM_HEADS = 8
_MAX_TIME_DIFF = 2592000.0
_B = 4096
_S = 50
_P = _S * _S

_SCALE = (_NUM_BUCKETS - 1) / np.log(_MAX_TIME_DIFF + 1.0)
_L = 2.0 / np.log(2.0)
_C0 = np.float32(_L)
_C1 = np.float32(_L / 3.0)
_C2 = np.float32(_L / 5.0)
_C3 = np.float32(_L / 7.0)
_C4 = np.float32(_L / 9.0)
_SQRT2 = np.float32(np.sqrt(2.0))
_LN2_SCALE = np.float32(np.log(2.0) * _SCALE)

_INFO = plsc.get_sparse_core_info()
_NW = _INFO.num_cores * _INFO.num_subcores
_ITEMS_PER_W = _B // _NW

_N_CHUNKS = 157
_TAIL_START = _P - 16


def _bucket_ids(d):
    y = d + jnp.float32(1.0)
    yi = lax.bitcast_convert_type(y, jnp.int32)
    e = lax.shift_right_arithmetic(yi, jnp.int32(23)) - jnp.int32(127)
    m = lax.bitcast_convert_type(
        (yi & jnp.int32(0x7FFFFF)) | jnp.int32(0x3F800000), jnp.float32)
    big = m > _SQRT2
    m = jnp.where(big, m * jnp.float32(0.5), m)
    ef = (e + big.astype(jnp.int32)).astype(jnp.float32)
    s = (m - jnp.float32(1.0)) / (m + jnp.float32(1.0))
    z = s * s
    p = _C4
    p = p * z + _C3
    p = p * z + _C2
    p = p * z + _C1
    p = p * z + _C0
    v = (ef + s * p) * _LN2_SCALE
    idx = v.astype(jnp.int32)
    return jnp.clip(idx, 0, _NUM_BUCKETS - 1)


def _sc_kernel(raw_hbm, tab_hbm, out_hbm, rawv, tabv, tbuf, outbuf):
    wid = lax.axis_index("s") * _INFO.num_cores + lax.axis_index("c")
    b0 = wid * _ITEMS_PER_W

    pltpu.sync_copy(tab_hbm, tabv)
    pltpu.sync_copy(raw_hbm.at[pl.ds(b0 * _S, _ITEMS_PER_W * _S)], rawv)

    iota = lax.iota(jnp.int32, 16)

    def item_body(k, carry):
        base = k * _S
        ch0 = plsc.load_gather(rawv, [base + iota])
        ch1 = plsc.load_gather(rawv, [base + 16 + iota])
        ch2 = plsc.load_gather(rawv, [base + 32 + iota])
        ch3 = plsc.load_gather(rawv, [jnp.minimum(base + 48 + iota,
                                                  jnp.int32(_ITEMS_PER_W * _S - 1))])
        t0 = plsc.cumsum(ch0)
        s0 = jnp.sum(ch0)
        t1 = plsc.cumsum(ch1) + s0
        s1 = s0 + jnp.sum(ch1)
        t2 = plsc.cumsum(ch2) + s1
        s2 = s1 + jnp.sum(ch2)
        t3 = plsc.cumsum(ch3) + s2
        tbuf[pl.ds(0, 16)] = t0
        tbuf[pl.ds(16, 16)] = t1
        tbuf[pl.ds(32, 16)] = t2
        tbuf[pl.ds(48, 16)] = t3

        def chunk_body(c, carry2):
            pbase = jnp.minimum(c * 16, jnp.int32(_TAIL_START))
            pv = pbase + iota
            iv = pv // jnp.int32(_S)
            jv = pv - iv * jnp.int32(_S)
            ti = plsc.load_gather(tbuf, [iv])
            tj = plsc.load_gather(tbuf, [jv])
            d = jnp.abs(ti - tj)
            idx8 = lax.shift_left(_bucket_ids(d), jnp.int32(3))
            for h in range(_NUM_HEADS):
                g = plsc.load_gather(tabv, [idx8 + jnp.int32(h)])
                plsc.store_scatter(outbuf, [pv + jnp.int32(h * _P)], g)
            return carry2

        lax.fori_loop(0, _N_CHUNKS, chunk_body, 0, unroll=2)
        pltpu.sync_copy(outbuf, out_hbm.at[b0 + k])
        return carry

    lax.fori_loop(0, _ITEMS_PER_W, item_body, 0)


def kernel(raw_time_diff, time_emb_weight):
    tab_flat = time_emb_weight.reshape(_NUM_BUCKETS * _NUM_HEADS)
    raw_flat = raw_time_diff.reshape(_B * _S)

    mesh = plsc.VectorSubcoreMesh(core_axis_name="c", subcore_axis_name="s")
    run = functools.partial(
        pl.kernel,
        mesh=mesh,
        compiler_params=pltpu.CompilerParams(needs_layout_passes=False),
        out_type=jax.ShapeDtypeStruct((_B, _NUM_HEADS * _P), jnp.float32),
        scratch_types=[
            pltpu.VMEM((_ITEMS_PER_W * _S,), jnp.float32),
            pltpu.VMEM((_NUM_BUCKETS * _NUM_HEADS,), jnp.float32),
            pltpu.VMEM((64,), jnp.float32),
            pltpu.VMEM((_NUM_HEADS * _P,), jnp.float32),
        ],
    )(_sc_kernel)
    out = run(raw_flat, tab_flat)
    return out.reshape(_B, _NUM_HEADS, _S, _S)


# speedup vs baseline: 116.0546x; 6.4366x over previous
import functools

import jax
import jax.numpy as jnp
import numpy as np
from jax import lax
from jax.experimental import pallas as pl
from jax.experimental.pallas import tpu as pltpu
from jax.experimental.pallas import tpu_sc as plsc

_NUM_BUCKETS = 128
_NUM_HEADS = 8
_MAX_TIME_DIFF = 2592000.0
_B = 4096
_S = 50

_SCALE = (_NUM_BUCKETS - 1) / np.log(_MAX_TIME_DIFF + 1.0)
_L = 2.0 / np.log(2.0)
_C0 = np.float32(_L)
_C1 = np.float32(_L / 3.0)
_C2 = np.float32(_L / 5.0)
_C3 = np.float32(_L / 7.0)
_C4 = np.float32(_L / 9.0)
_SQRT2 = np.float32(np.sqrt(2.0))
_LN2_SCALE = np.float32(np.log(2.0) * _SCALE)

_INFO = plsc.get_sparse_core_info()
_NW = _INFO.num_cores * _INFO.num_subcores
_BW = _B // _NW
_NG = _BW // 16


def _bucket_ids(d):
    y = d + jnp.float32(1.0)
    yi = lax.bitcast_convert_type(y, jnp.int32)
    e = lax.shift_right_arithmetic(yi, jnp.int32(23)) - jnp.int32(127)
    m = lax.bitcast_convert_type(
        (yi & jnp.int32(0x7FFFFF)) | jnp.int32(0x3F800000), jnp.float32)
    big = m > _SQRT2
    m = jnp.where(big, m * jnp.float32(0.5), m)
    ef = (e + big.astype(jnp.int32)).astype(jnp.float32)
    s = (m - jnp.float32(1.0)) / (m + jnp.float32(1.0))
    z = s * s
    p = _C4
    p = p * z + _C3
    p = p * z + _C2
    p = p * z + _C1
    p = p * z + _C0
    v = (ef + s * p) * _LN2_SCALE
    idx = v.astype(jnp.int32)
    return jnp.clip(idx, 0, _NUM_BUCKETS - 1)


def _sc_kernel(rawt_hbm, tab_hbm, out_hbm, rawtv, tabv, ttv,
               outbuf0, outbuf1, sem0, sem1):
    wid = lax.axis_index("s") * _INFO.num_cores + lax.axis_index("c")
    b0 = wid * _BW
    rt = wid // 8
    xt = wid - rt * 8

    pltpu.sync_copy(tab_hbm, tabv)
    pltpu.sync_copy(rawt_hbm.at[:, pl.ds(b0, _BW)], rawtv)

    accs = tuple(rawtv[0, pl.ds(16 * g, 16)] for g in range(_NG))
    for g in range(_NG):
        ttv[0, pl.ds(16 * g, 16)] = accs[g]

    def cum_body(i, accs):
        new = tuple(accs[g] + rawtv[i, pl.ds(16 * g, 16)] for g in range(_NG))
        for g in range(_NG):
            ttv[i, pl.ds(16 * g, 16)] = new[g]
        return new

    lax.fori_loop(1, _S, cum_body, accs)

    outbufs = (outbuf0, outbuf1)
    sems = (sem0, sem1)

    def win_body(g, carry):
        for r in range(2):
            i = g * 2 + r
            outbuf = outbufs[r]
            sem = sems[r]

            dst = out_hbm.at[pl.ds(i * _S, _S),
                             pl.ds(rt * _NUM_HEADS, _NUM_HEADS),
                             pl.ds(xt * 128, 128)]

            @pl.when(g > 0)
            def _wait_prev():
                pltpu.make_async_copy(outbuf, dst, sem).wait()

            tis = tuple(ttv[i, pl.ds(16 * gg, 16)] for gg in range(_NG))

            @plsc.parallel_loop(0, _S, unroll=1)
            def _pair(j):
                for gg in range(_NG):
                    tj = ttv[j, pl.ds(16 * gg, 16)]
                    d = jnp.abs(tis[gg] - tj)
                    idx8 = lax.shift_left(_bucket_ids(d), jnp.int32(3))
                    for h in range(_NUM_HEADS):
                        val = plsc.load_gather(tabv, [idx8 + jnp.int32(h)])
                        outbuf[j, h, pl.ds(16 * gg, 16)] = val

            pltpu.async_copy(outbuf, dst, sem)
        return carry

    lax.fori_loop(0, _S // 2, win_body, 0)
    drain = out_hbm.at[pl.ds(0, _S), pl.ds(rt * _NUM_HEADS, _NUM_HEADS),
                       pl.ds(xt * 128, 128)]
    pltpu.make_async_copy(outbuf0, drain, sem0).wait()
    pltpu.make_async_copy(outbuf1, drain, sem1).wait()


def kernel(raw_time_diff, time_emb_weight):
    rawt = raw_time_diff.T
    tab_flat = time_emb_weight.reshape(_NUM_BUCKETS * _NUM_HEADS)

    mesh = plsc.VectorSubcoreMesh(core_axis_name="c", subcore_axis_name="s")
    run = functools.partial(
        pl.kernel,
        mesh=mesh,
        compiler_params=pltpu.CompilerParams(needs_layout_passes=False),
        out_type=jax.ShapeDtypeStruct((_S * _S, _NW, _NUM_HEADS * _BW),
                                      jnp.float32),
        scratch_types=[
            pltpu.VMEM((_S, _BW), jnp.float32),
            pltpu.VMEM((_NUM_BUCKETS * _NUM_HEADS,), jnp.float32),
            pltpu.VMEM((_S, _BW), jnp.float32),
            pltpu.VMEM((_S, _NUM_HEADS, _BW), jnp.float32),
            pltpu.VMEM((_S, _NUM_HEADS, _BW), jnp.float32),
            pltpu.SemaphoreType.DMA,
            pltpu.SemaphoreType.DMA,
        ],
    )(_sc_kernel)
    out = run(rawt, tab_flat)
    out = out.reshape(_S, _S, 4, _NUM_HEADS, 8 * 128)
    out = out.transpose(2, 4, 3, 0, 1).reshape(_B, _NUM_HEADS, _S, _S)
    return out
